# baseline (speedup 1.0000x reference)
import jax
import jax.numpy as jnp
from jax.experimental import pallas as pl
from jax.experimental.pallas import tpu as pltpu

_NUM_CENTROIDS = 8
_LOSS_SCALE = 1.25


def _vq_block_kernel(x_ref, w_ref, b_ref, cb_ref,
                     q_ref, idx_ref, tw_ref, enc_ref, loss_ref):
    x = x_ref[...]
    logits = jnp.dot(x, w_ref[...],
                     preferred_element_type=jnp.float32) + b_ref[...]
    k = logits.shape[1]
    nc = 4
    ck = k // nc
    iota_f = jax.lax.broadcasted_iota(
        jnp.int32, logits.shape, 1).astype(jnp.float32)
    iota_c = iota_f[:, :ck]
    ckf = jnp.float32(ck)

    cand_v, cand_i = [], []
    for c in range(nc):
        vc = logits[:, c * ck:(c + 1) * ck]
        for h in range(_NUM_CENTROIDS):
            m = jnp.max(vc, axis=1, keepdims=True)
            am = jnp.min(jnp.where(vc == m, iota_c, ckf), axis=1, keepdims=True)
            cand_v.append(m)
            cand_i.append(am + jnp.float32(c * ck))
            if h < _NUM_CENTROIDS - 1:
                vc = jnp.where(iota_c == am, -jnp.inf, vc)
    cv = jnp.concatenate(cand_v, axis=1)
    ci = jnp.concatenate(cand_i, axis=1)
    ncand = nc * _NUM_CENTROIDS
    iota_m = jax.lax.broadcasted_iota(
        jnp.int32, cv.shape, 1).astype(jnp.float32)
    top_v, top_i = [], []
    for h in range(_NUM_CENTROIDS):
        m = jnp.max(cv, axis=1, keepdims=True)
        pos = jnp.min(jnp.where(cv == m, iota_m, jnp.float32(ncand)),
                      axis=1, keepdims=True)
        gi = jnp.min(jnp.where(iota_m == pos, ci, jnp.float32(k)),
                     axis=1, keepdims=True)
        top_v.append(m)
        top_i.append(gi)
        if h < _NUM_CENTROIDS - 1:
            cv = jnp.where(iota_m == pos, -jnp.inf, cv)
    tv = jnp.concatenate(top_v, axis=1)
    tif = jnp.concatenate(top_i, axis=1)

    e = jnp.exp(tv - tv[:, :1])
    tw = e / jnp.sum(e, axis=1, keepdims=True)

    idx_ref[...] = tif.astype(jnp.int32)
    tw_ref[...] = tw

    enc = jnp.zeros_like(logits)
    for h in range(_NUM_CENTROIDS):
        enc += jnp.where(iota_f == tif[:, h:h + 1], tw[:, h:h + 1], 0.0)
    enc_ref[...] = enc

    q = jnp.dot(enc, cb_ref[...], preferred_element_type=jnp.float32)
    q_ref[...] = q

    loss_ref[...] = jnp.sum((q - x) ** 2).reshape(1, 1, 1)


def kernel(x, W, b, codebook):
    B, T, D = x.shape
    K = codebook.shape[0]
    N = B * T
    BLK = 512
    grid = N // BLK
    xf = x.reshape(N, D)

    q, ti, tw, enc, loss = pl.pallas_call(
        _vq_block_kernel,
        grid=(grid,),
        in_specs=[
            pl.BlockSpec((BLK, D), lambda i: (i, 0)),
            pl.BlockSpec((D, K), lambda i: (0, 0)),
            pl.BlockSpec((K,), lambda i: (0,)),
            pl.BlockSpec((K, D), lambda i: (0, 0)),
        ],
        out_specs=[
            pl.BlockSpec((BLK, D), lambda i: (i, 0)),
            pl.BlockSpec((BLK, _NUM_CENTROIDS), lambda i: (i, 0)),
            pl.BlockSpec((BLK, _NUM_CENTROIDS), lambda i: (i, 0)),
            pl.BlockSpec((BLK, K), lambda i: (i, 0)),
            pl.BlockSpec((1, 1, 1), lambda i: (i, 0, 0)),
        ],
        out_shape=[
            jax.ShapeDtypeStruct((N, D), jnp.float32),
            jax.ShapeDtypeStruct((N, _NUM_CENTROIDS), jnp.int32),
            jax.ShapeDtypeStruct((N, _NUM_CENTROIDS), jnp.float32),
            jax.ShapeDtypeStruct((N, K), jnp.float32),
            jax.ShapeDtypeStruct((grid, 1, 1), jnp.float32),
        ],
        compiler_params=pltpu.CompilerParams(
            dimension_semantics=("parallel",),
        ),
    )(xf, W, b, codebook)

    quantized_st = q.reshape(B, T, D)
    top_indices = ti.reshape(B, T, _NUM_CENTROIDS)
    top_weights = tw.reshape(B, T, _NUM_CENTROIDS)
    encodings = enc.reshape(B, T, K)
    loss_out = (jnp.sum(loss) * _LOSS_SCALE) / (N * D)
    return (quantized_st, top_indices, top_weights, encodings, loss_out)


# speedup vs baseline: 1.4374x; 1.4374x over previous
import jax
import jax.numpy as jnp
from jax.experimental import pallas as pl
from jax.experimental.pallas import tpu as pltpu

_NUM_CENTROIDS = 8
_LOSS_SCALE = 1.25


def _vq_block_kernel(x_ref, w_ref, b_ref, cb_ref,
                     q_ref, idx_ref, tw_ref, enc_ref, loss_ref):
    x = x_ref[...]
    logits = jnp.dot(x, w_ref[...],
                     preferred_element_type=jnp.float32) + b_ref[...]
    k = logits.shape[1]
    iota_f = jax.lax.broadcasted_iota(
        jnp.int32, logits.shape, 1).astype(jnp.float32)
    kf = jnp.float32(k)

    vals = logits
    top_v, top_i = [], []
    for h in range(_NUM_CENTROIDS):
        m = jnp.max(vals, axis=1, keepdims=True)
        am = jnp.min(jnp.where(vals == m, iota_f, kf), axis=1, keepdims=True)
        top_v.append(m)
        top_i.append(am)
        if h < _NUM_CENTROIDS - 1:
            vals = jnp.where(iota_f == am, -jnp.inf, vals)
    tv = jnp.concatenate(top_v, axis=1)
    tif = jnp.concatenate(top_i, axis=1)

    e = jnp.exp(tv - tv[:, :1])
    tw = e / jnp.sum(e, axis=1, keepdims=True)

    idx_ref[...] = tif.astype(jnp.int32)
    tw_ref[...] = tw

    enc = jnp.zeros_like(logits)
    for h in range(_NUM_CENTROIDS):
        enc += jnp.where(iota_f == tif[:, h:h + 1], tw[:, h:h + 1], 0.0)
    enc_ref[...] = enc

    q = jnp.dot(enc, cb_ref[...], preferred_element_type=jnp.float32)
    q_ref[...] = q

    loss_ref[...] = jnp.sum((q - x) ** 2).reshape(1, 1, 1)


def kernel(x, W, b, codebook):
    B, T, D = x.shape
    K = codebook.shape[0]
    N = B * T
    BLK = 1024
    grid = N // BLK
    xf = x.reshape(N, D)

    q, ti, tw, enc, loss = pl.pallas_call(
        _vq_block_kernel,
        grid=(grid,),
        in_specs=[
            pl.BlockSpec((BLK, D), lambda i: (i, 0)),
            pl.BlockSpec((D, K), lambda i: (0, 0)),
            pl.BlockSpec((K,), lambda i: (0,)),
            pl.BlockSpec((K, D), lambda i: (0, 0)),
        ],
        out_specs=[
            pl.BlockSpec((BLK, D), lambda i: (i, 0)),
            pl.BlockSpec((BLK, _NUM_CENTROIDS), lambda i: (i, 0)),
            pl.BlockSpec((BLK, _NUM_CENTROIDS), lambda i: (i, 0)),
            pl.BlockSpec((BLK, K), lambda i: (i, 0)),
            pl.BlockSpec((1, 1, 1), lambda i: (i, 0, 0)),
        ],
        out_shape=[
            jax.ShapeDtypeStruct((N, D), jnp.float32),
            jax.ShapeDtypeStruct((N, _NUM_CENTROIDS), jnp.int32),
            jax.ShapeDtypeStruct((N, _NUM_CENTROIDS), jnp.float32),
            jax.ShapeDtypeStruct((N, K), jnp.float32),
            jax.ShapeDtypeStruct((grid, 1, 1), jnp.float32),
        ],
        compiler_params=pltpu.CompilerParams(
            dimension_semantics=("parallel",),
        ),
    )(xf, W, b, codebook)

    quantized_st = q.reshape(B, T, D)
    top_indices = ti.reshape(B, T, _NUM_CENTROIDS)
    top_weights = tw.reshape(B, T, _NUM_CENTROIDS)
    encodings = enc.reshape(B, T, K)
    loss_out = (jnp.sum(loss) * _LOSS_SCALE) / (N * D)
    return (quantized_st, top_indices, top_weights, encodings, loss_out)


# speedup vs baseline: 1.4433x; 1.0041x over previous
import jax
import jax.numpy as jnp
from jax.experimental import pallas as pl
from jax.experimental.pallas import tpu as pltpu

_NUM_CENTROIDS = 8
_LOSS_SCALE = 1.25


def _vq_block_kernel(x_ref, w_ref, b_ref, cb_ref,
                     q_ref, idx_ref, tw_ref, enc_ref, loss_ref):
    x = x_ref[...]
    logits = jnp.dot(x, w_ref[...],
                     preferred_element_type=jnp.float32) + b_ref[...]
    k = logits.shape[1]
    iota_f = jax.lax.broadcasted_iota(
        jnp.int32, logits.shape, 1).astype(jnp.float32)
    kf = jnp.float32(k)

    vals = logits
    top_v, top_i = [], []
    for h in range(_NUM_CENTROIDS):
        m = jnp.max(vals, axis=1, keepdims=True)
        am = jnp.min(jnp.where(vals == m, iota_f, kf), axis=1, keepdims=True)
        top_v.append(m)
        top_i.append(am)
        if h < _NUM_CENTROIDS - 1:
            vals = jnp.where(iota_f == am, -jnp.inf, vals)
    tv = jnp.concatenate(top_v, axis=1)
    tif = jnp.concatenate(top_i, axis=1)

    e = jnp.exp(tv - tv[:, :1])
    tw = e / jnp.sum(e, axis=1, keepdims=True)

    idx_ref[...] = tif.astype(jnp.int32)
    tw_ref[...] = tw

    enc = jnp.zeros_like(logits)
    for h in range(_NUM_CENTROIDS):
        enc += jnp.where(iota_f == tif[:, h:h + 1], tw[:, h:h + 1], 0.0)
    enc_ref[...] = enc

    q = jnp.dot(enc, cb_ref[...], preferred_element_type=jnp.float32)
    q_ref[...] = q

    loss_ref[...] = jnp.sum((q - x) ** 2).reshape(1, 1, 1)


def kernel(x, W, b, codebook):
    B, T, D = x.shape
    K = codebook.shape[0]
    N = B * T
    BLK = 2304
    grid = N // BLK
    xf = x.reshape(N, D)

    q, ti, tw, enc, loss = pl.pallas_call(
        _vq_block_kernel,
        grid=(grid,),
        in_specs=[
            pl.BlockSpec((BLK, D), lambda i: (i, 0)),
            pl.BlockSpec((D, K), lambda i: (0, 0)),
            pl.BlockSpec((K,), lambda i: (0,)),
            pl.BlockSpec((K, D), lambda i: (0, 0)),
        ],
        out_specs=[
            pl.BlockSpec((BLK, D), lambda i: (i, 0)),
            pl.BlockSpec((BLK, _NUM_CENTROIDS), lambda i: (i, 0)),
            pl.BlockSpec((BLK, _NUM_CENTROIDS), lambda i: (i, 0)),
            pl.BlockSpec((BLK, K), lambda i: (i, 0)),
            pl.BlockSpec((1, 1, 1), lambda i: (i, 0, 0)),
        ],
        out_shape=[
            jax.ShapeDtypeStruct((N, D), jnp.float32),
            jax.ShapeDtypeStruct((N, _NUM_CENTROIDS), jnp.int32),
            jax.ShapeDtypeStruct((N, _NUM_CENTROIDS), jnp.float32),
            jax.ShapeDtypeStruct((N, K), jnp.float32),
            jax.ShapeDtypeStruct((grid, 1, 1), jnp.float32),
        ],
        compiler_params=pltpu.CompilerParams(
            dimension_semantics=("parallel",),
        ),
    )(xf, W, b, codebook)

    quantized_st = q.reshape(B, T, D)
    top_indices = ti.reshape(B, T, _NUM_CENTROIDS)
    top_weights = tw.reshape(B, T, _NUM_CENTROIDS)
    encodings = enc.reshape(B, T, K)
    loss_out = (jnp.sum(loss) * _LOSS_SCALE) / (N * D)
    return (quantized_st, top_indices, top_weights, encodings, loss_out)
